# Initial kernel scaffold; baseline (speedup 1.0000x reference)
import jax
import jax.numpy as jnp
from jax.experimental import pallas as pl


def kernel(edge_index, c_ids, u, batch, num_tuples, rel_ids, concept_table, rel_table, edge_mlp, node_mlp1, node_mlp2):
    raise NotImplementedError("write your pallas kernel here")


# speedup vs baseline: 2.0946x; 2.0946x over previous
import functools

import jax
import jax.numpy as jnp
from jax import lax
from jax.experimental import pallas as pl
from jax.experimental.pallas import tpu as pltpu
from jax.experimental.pallas import tpu_sc as plsc

N_NODES = 10000
N_EDGES = 320000
D = 128

W = 128
C_PAD = 12288
NODE_PAD = 10240
EP = 327680
EDGE_GRID = EP // W
BE = 2560
DUMMY = N_NODES
R_PAD = 64


def _mesh():
    return plsc.VectorSubcoreMesh(core_axis_name="c", subcore_axis_name="s")


def _sc_embed_gather(concept_table, c_ids_pad2d):
    @functools.partial(
        pl.kernel,
        out_type=jax.ShapeDtypeStruct((C_PAD, D), jnp.float32),
        mesh=_mesh(),
    )
    def k(ct_hbm, cid_hbm, na_hbm):
        def cbody(i_v, o_v):
            pltpu.sync_copy(ct_hbm.at[i_v.at[0]], o_v)

        pltpu.emit_pipeline(
            cbody,
            grid=(C_PAD // W,),
            in_specs=[pl.BlockSpec((1, W), lambda i: (0, i))],
            out_specs=[pl.BlockSpec((W, D), lambda i: (i, 0))],
            core_axis_name=("c", "s"),
            dimension_semantics=(pltpu.PARALLEL,),
        )(cid_hbm, na_hbm)

    return k(concept_table, c_ids_pad2d)


def _sc_edge_gather(ac_table, b_table, row2d, col2d):
    @functools.partial(
        pl.kernel,
        out_type=jax.ShapeDtypeStruct((EP, 2 * D), jnp.float32),
        mesh=_mesh(),
    )
    def ka(ac_hbm, row_hbm, ga_hbm):
        def abody(ir_v, ga_v):
            pltpu.sync_copy(ac_hbm.at[ir_v.at[0]], ga_v)

        pltpu.emit_pipeline(
            abody,
            grid=(EDGE_GRID,),
            in_specs=[pl.BlockSpec((1, W), lambda i: (0, i))],
            out_specs=[pl.BlockSpec((W, 2 * D), lambda i: (i, 0))],
            core_axis_name=("c", "s"),
            dimension_semantics=(pltpu.PARALLEL,),
        )(row_hbm, ga_hbm)

    @functools.partial(
        pl.kernel,
        out_type=jax.ShapeDtypeStruct((EP, D), jnp.float32),
        mesh=_mesh(),
    )
    def kb(b_hbm, col_hbm, gb_hbm):
        def bbody(ic_v, gb_v):
            pltpu.sync_copy(b_hbm.at[ic_v.at[0]], gb_v)

        pltpu.emit_pipeline(
            bbody,
            grid=(EDGE_GRID,),
            in_specs=[pl.BlockSpec((1, W), lambda i: (0, i))],
            out_specs=[pl.BlockSpec((W, D), lambda i: (i, 0))],
            core_axis_name=("c", "s"),
            dimension_semantics=(pltpu.PARALLEL,),
        )(col_hbm, gb_hbm)

    return ka(ac_table, row2d), kb(b_table, col2d)


SUBCH = NODE_PAD // 16 // W


def _zero_acc(z_hbm, acc, sid):
    @pl.loop(0, SUBCH)
    def _(j):
        base = sid * (SUBCH * W) + j * W
        pltpu.sync_copy(z_hbm, acc.at[pl.ds(base, W)])


def _writeback(acc, dst_hbm, sid):
    @pl.loop(0, SUBCH)
    def _(j):
        base = sid * (SUBCH * W) + j * W
        sl = pl.ds(base, W)
        pltpu.sync_copy(acc.at[sl], dst_hbm.at[sl])


def _sc_scatter_msg(msg0, msg1, col2d, zeros):
    @functools.partial(
        pl.kernel,
        out_type=(
            jax.ShapeDtypeStruct((NODE_PAD, D), jnp.float32),
            jax.ShapeDtypeStruct((NODE_PAD, D), jnp.float32),
        ),
        mesh=_mesh(),
        scratch_types=[pltpu.VMEM_SHARED((NODE_PAD, D), jnp.float32)],
    )
    def k(m0_hbm, m1_hbm, col_hbm, z_hbm, s0_hbm, s1_hbm, acc):
        cid = lax.axis_index("c")
        sid = lax.axis_index("s")

        _zero_acc(z_hbm, acc, sid)
        plsc.subcore_barrier()

        def addmsg(i_v, m_v):
            pltpu.sync_copy(m_v, acc.at[i_v.at[0]], add=True)

        mspecs = dict(
            grid=(EDGE_GRID,),
            in_specs=[
                pl.BlockSpec((1, W), lambda i: (0, i)),
                pl.BlockSpec((W, D), lambda i: (i, 0)),
            ],
            out_specs=[],
            core_axis_name="s",
            dimension_semantics=(pltpu.PARALLEL,),
        )

        @pl.when(cid == 0)
        def _():
            pltpu.emit_pipeline(addmsg, **mspecs)(col_hbm, m0_hbm)

        @pl.when(cid == 1)
        def _():
            pltpu.emit_pipeline(addmsg, **mspecs)(col_hbm, m1_hbm)

        plsc.subcore_barrier()

        @pl.when(cid == 0)
        def _():
            _writeback(acc, s0_hbm, sid)

        @pl.when(cid == 1)
        def _():
            _writeback(acc, s1_hbm, sid)

    return k(msg0, msg1, col2d, zeros)


def _sc_counts(col_lo, col_hi, zeros, ones):
    @functools.partial(
        pl.kernel,
        out_type=(
            jax.ShapeDtypeStruct((NODE_PAD, D), jnp.float32),
            jax.ShapeDtypeStruct((NODE_PAD, D), jnp.float32),
        ),
        mesh=_mesh(),
        scratch_types=[
            pltpu.VMEM_SHARED((NODE_PAD, D), jnp.float32),
            pltpu.VMEM((W, D), jnp.float32),
        ],
    )
    def k(clo_hbm, chi_hbm, z_hbm, o_hbm, c0_hbm, c1_hbm, acc, obuf):
        cid = lax.axis_index("c")
        sid = lax.axis_index("s")

        _zero_acc(z_hbm, acc, sid)
        pltpu.sync_copy(o_hbm, obuf)
        plsc.subcore_barrier()

        def addones(i_v):
            pltpu.sync_copy(obuf, acc.at[i_v.at[0]], add=True)

        ospecs = dict(
            grid=(EDGE_GRID // 2,),
            in_specs=[pl.BlockSpec((1, W), lambda i: (0, i))],
            out_specs=[],
            core_axis_name="s",
            dimension_semantics=(pltpu.PARALLEL,),
        )

        @pl.when(cid == 0)
        def _():
            pltpu.emit_pipeline(addones, **ospecs)(clo_hbm)

        @pl.when(cid == 1)
        def _():
            pltpu.emit_pipeline(addones, **ospecs)(chi_hbm)

        plsc.subcore_barrier()

        @pl.when(cid == 0)
        def _():
            _writeback(acc, c0_hbm, sid)

        @pl.when(cid == 1)
        def _():
            _writeback(acc, c1_hbm, sid)

    return k(col_lo, col_hi, zeros, ones)


def _ln(x, g, b):
    mu = jnp.mean(x, axis=-1, keepdims=True)
    var = jnp.mean((x - mu) ** 2, axis=-1, keepdims=True)
    return (x - mu) / jnp.sqrt(var + 1e-5) * g + b


def _dot(a, b):
    return jnp.dot(a, b, preferred_element_type=jnp.float32)


def _tc_project(node_attr, w1a, w1b, v1a, relt_pad, w1c):
    def body(na_ref, wa_ref, wb_ref, va_ref, rt_ref, wc_ref,
             ac_ref, b_ref, rw_ref):
        na = na_ref[...]
        ac_ref[:, :D] = _dot(na, wa_ref[...])
        ac_ref[:, D:] = _dot(na, va_ref[...])
        b_ref[...] = _dot(na, wb_ref[...])
        rw_ref[...] = _dot(rt_ref[...], wc_ref[...])

    return pl.pallas_call(
        body,
        out_shape=(
            jax.ShapeDtypeStruct((NODE_PAD, 2 * D), jnp.float32),
            jax.ShapeDtypeStruct((NODE_PAD, D), jnp.float32),
            jax.ShapeDtypeStruct((R_PAD, D), jnp.float32),
        ),
    )(node_attr, w1a, w1b, v1a, relt_pad, w1c)


def _tc_edge(ga, gb, rid, relw, ew):
    def body(ga_ref, gb_ref, rid_ref, rw_ref,
             eb1_r, eg1_r, ebe1_r, ew2_r, eb2_r, eg2_r, ebe2_r,
             ew3_r, eb3_r, v1b_r, nb1_r, ng1_r, nbe1_r, nv2_r, nb2_r,
             ng2_r, nbe2_r, nv3_r, nb3_r, eo_ref, m0_ref, m1_ref):
        oh = (rid_ref[...] == lax.broadcasted_iota(
            jnp.int32, (BE, R_PAD), 1)).astype(jnp.float32)
        h = ga_ref[:, :D] + gb_ref[...] + _dot(oh, rw_ref[...]) + eb1_r[...]
        h = jax.nn.gelu(_ln(h, eg1_r[...], ebe1_r[...]))
        h = _dot(h, ew2_r[...]) + eb2_r[...]
        h = jax.nn.gelu(_ln(h, eg2_r[...], ebe2_r[...]))
        e = _dot(h, ew3_r[...]) + eb3_r[...]
        eo_ref[...] = e
        m = ga_ref[:, D:] + _dot(e, v1b_r[...]) + nb1_r[...]
        m = jax.nn.gelu(_ln(m, ng1_r[...], nbe1_r[...]))
        m = _dot(m, nv2_r[...]) + nb2_r[...]
        m = jax.nn.gelu(_ln(m, ng2_r[...], nbe2_r[...]))
        m0_ref[...] = _dot(m, nv3_r[:, :D]) + nb3_r[:, :D]
        m1_ref[...] = _dot(m, nv3_r[:, D:]) + nb3_r[:, D:]

    full = lambda shape: pl.BlockSpec(shape, lambda i: tuple(0 for _ in shape))
    wspecs = [full(w.shape) for w in ew]
    return pl.pallas_call(
        body,
        grid=(N_EDGES // BE,),
        in_specs=[
            pl.BlockSpec((BE, 2 * D), lambda i: (i, 0)),
            pl.BlockSpec((BE, D), lambda i: (i, 0)),
            pl.BlockSpec((BE, 1), lambda i: (i, 0)),
            full((R_PAD, D)),
        ] + wspecs,
        out_specs=[
            pl.BlockSpec((BE, D), lambda i: (i, 0)),
            pl.BlockSpec((BE, D), lambda i: (i, 0)),
            pl.BlockSpec((BE, D), lambda i: (i, 0)),
        ],
        out_shape=(
            jax.ShapeDtypeStruct((N_EDGES, D), jnp.float32),
            jax.ShapeDtypeStruct((EP, D), jnp.float32),
            jax.ShapeDtypeStruct((EP, D), jnp.float32),
        ),
    )(ga, gb, rid, relw, *ew)


def _tc_node(node_attr, s0, s1, c0, c1, nw):
    def body(na_ref, s0_ref, s1_ref, c0_ref, c1_ref,
             u1a_r, u1b0_r, u1b1_r, ub1_r, ug1_r, ube1_r,
             u2_r, ub2_r, ug2_r, ube2_r, u3_r, ub3_r, o_ref):
        c = jnp.clip(c0_ref[...][:, 0:1] + c1_ref[...][:, 0:1], 1.0, None)
        m0 = s0_ref[...] / c
        m1 = s1_ref[...] / c
        h = (_dot(na_ref[...], u1a_r[...]) + _dot(m0, u1b0_r[...])
             + _dot(m1, u1b1_r[...]) + ub1_r[...])
        h = jax.nn.gelu(_ln(h, ug1_r[...], ube1_r[...]))
        h = _dot(h, u2_r[...]) + ub2_r[...]
        h = jax.nn.gelu(_ln(h, ug2_r[...], ube2_r[...]))
        o_ref[...] = _dot(h, u3_r[...]) + ub3_r[...]

    return pl.pallas_call(
        body,
        out_shape=jax.ShapeDtypeStruct((NODE_PAD, D), jnp.float32),
    )(node_attr, s0, s1, c0, c1, *nw)


def kernel(edge_index, c_ids, u, batch, num_tuples, rel_ids,
           concept_table, rel_table, edge_mlp, node_mlp1, node_mlp2):
    row = edge_index[0]
    col = edge_index[1]

    (eW1, eb1, eg1, ebe1), (eW2, eb2, eg2, ebe2), (eW3, eb3, _, _) = edge_mlp
    (nW1, nb1, ng1, nbe1), (nW2, nb2, ng2, nbe2), (nW3, nb3, _, _) = node_mlp1
    (uW1, ub1, ug1, ube1), (uW2, ub2, ug2, ube2), (uW3, ub3, _, _) = node_mlp2

    r2 = lambda v: v.reshape(1, -1)
    epad = lambda v, c: jnp.pad(v, (0, EP - N_EDGES),
                                constant_values=c).reshape(1, EP)

    c_pad = jnp.pad(c_ids, (0, C_PAD - N_NODES)).reshape(1, C_PAD)
    na_pad = _sc_embed_gather(concept_table, c_pad)
    node_attr = na_pad[:NODE_PAD]

    relt_pad = jnp.pad(rel_table, ((0, R_PAD - rel_table.shape[0]), (0, 0)))
    ac, bd, relw = _tc_project(node_attr, eW1[:D], eW1[D:2 * D], nW1[:D],
                               relt_pad, eW1[2 * D:])
    row2d = epad(row, DUMMY)
    col2d = epad(col, DUMMY)
    ga, gb = _sc_edge_gather(ac, bd, row2d, col2d)

    ew = (r2(eb1), r2(eg1), r2(ebe1), eW2, r2(eb2), r2(eg2),
          r2(ebe2), eW3, r2(eb3), nW1[D:], r2(nb1), r2(ng1), r2(nbe1),
          nW2, r2(nb2), r2(ng2), r2(nbe2), nW3, r2(nb3))
    eout, m0, m1 = _tc_edge(ga, gb, rel_ids.reshape(N_EDGES, 1), relw, ew)

    zeros = jnp.zeros((W, D), jnp.float32)
    c0, c1 = _sc_counts(col2d[:, :EP // 2], col2d[:, EP // 2:],
                        zeros, jnp.ones((W, D), jnp.float32))
    s0, s1 = _sc_scatter_msg(m0, m1, col2d, zeros)

    nw = (uW1[:D], uW1[D:2 * D], uW1[2 * D:], r2(ub1), r2(ug1), r2(ube1),
          uW2, r2(ub2), r2(ug2), r2(ube2), uW3, r2(ub3))
    x_new = _tc_node(node_attr, s0, s1, c0, c1, nw)

    return (x_new[:N_NODES], eout, u)


# speedup vs baseline: 2.2066x; 1.0535x over previous
import functools

import jax
import jax.numpy as jnp
from jax import lax
from jax.experimental import pallas as pl
from jax.experimental.pallas import tpu as pltpu
from jax.experimental.pallas import tpu_sc as plsc

N_NODES = 10000
N_EDGES = 320000
D = 128

W = 128
C_PAD = 12288
NODE_PAD = 10240
EP = 327680
EDGE_GRID = EP // W
BE = 2560
DUMMY = N_NODES
R_PAD = 64


def _mesh():
    return plsc.VectorSubcoreMesh(core_axis_name="c", subcore_axis_name="s")


def _sc_embed_gather(concept_table, c_ids_pad2d):
    @functools.partial(
        pl.kernel,
        out_type=jax.ShapeDtypeStruct((C_PAD, D), jnp.float32),
        mesh=_mesh(),
    )
    def k(ct_hbm, cid_hbm, na_hbm):
        def cbody(i_v, o_v):
            pltpu.sync_copy(ct_hbm.at[i_v.at[0]], o_v)

        pltpu.emit_pipeline(
            cbody,
            grid=(C_PAD // W,),
            in_specs=[pl.BlockSpec((1, W), lambda i: (0, i))],
            out_specs=[pl.BlockSpec((W, D), lambda i: (i, 0))],
            core_axis_name=("c", "s"),
            dimension_semantics=(pltpu.PARALLEL,),
        )(cid_hbm, na_hbm)

    return k(concept_table, c_ids_pad2d)


def _sc_edge_gather(ac_table, b_table, row2d, col2d):
    @functools.partial(
        pl.kernel,
        out_type=jax.ShapeDtypeStruct((EP, D), jnp.int32),
        mesh=_mesh(),
    )
    def ka(ac_hbm, row_hbm, ga_hbm):
        def abody(ir_v, ga_v):
            pltpu.sync_copy(ac_hbm.at[ir_v.at[0]], ga_v)

        pltpu.emit_pipeline(
            abody,
            grid=(EDGE_GRID,),
            in_specs=[pl.BlockSpec((1, W), lambda i: (0, i))],
            out_specs=[pl.BlockSpec((W, D), lambda i: (i, 0))],
            core_axis_name=("c", "s"),
            dimension_semantics=(pltpu.PARALLEL,),
        )(row_hbm, ga_hbm)

    @functools.partial(
        pl.kernel,
        out_type=jax.ShapeDtypeStruct((EP, D), jnp.float32),
        mesh=_mesh(),
    )
    def kb(b_hbm, col_hbm, gb_hbm):
        def bbody(ic_v, gb_v):
            pltpu.sync_copy(b_hbm.at[ic_v.at[0]], gb_v)

        pltpu.emit_pipeline(
            bbody,
            grid=(EDGE_GRID,),
            in_specs=[pl.BlockSpec((1, W), lambda i: (0, i))],
            out_specs=[pl.BlockSpec((W, D), lambda i: (i, 0))],
            core_axis_name=("c", "s"),
            dimension_semantics=(pltpu.PARALLEL,),
        )(col_hbm, gb_hbm)

    return ka(ac_table, row2d), kb(b_table, col2d)


SUBCH = NODE_PAD // 16 // W


def _zero_acc(z_hbm, acc, sid):
    @pl.loop(0, SUBCH)
    def _(j):
        base = sid * (SUBCH * W) + j * W
        pltpu.sync_copy(z_hbm, acc.at[pl.ds(base, W)])


def _writeback(acc, dst_hbm, sid):
    @pl.loop(0, SUBCH)
    def _(j):
        base = sid * (SUBCH * W) + j * W
        sl = pl.ds(base, W)
        pltpu.sync_copy(acc.at[sl], dst_hbm.at[sl])


def _sc_scatter_msg(msg0, msg1, col2d, zeros):
    @functools.partial(
        pl.kernel,
        out_type=(
            jax.ShapeDtypeStruct((NODE_PAD, D), jnp.float32),
            jax.ShapeDtypeStruct((NODE_PAD, D), jnp.float32),
        ),
        mesh=_mesh(),
        scratch_types=[pltpu.VMEM_SHARED((NODE_PAD, D), jnp.float32)],
    )
    def k(m0_hbm, m1_hbm, col_hbm, z_hbm, s0_hbm, s1_hbm, acc):
        cid = lax.axis_index("c")
        sid = lax.axis_index("s")

        _zero_acc(z_hbm, acc, sid)
        plsc.subcore_barrier()

        def addmsg(i_v, m_v):
            pltpu.sync_copy(m_v, acc.at[i_v.at[0]], add=True)

        mspecs = dict(
            grid=(EDGE_GRID,),
            in_specs=[
                pl.BlockSpec((1, W), lambda i: (0, i)),
                pl.BlockSpec((W, D), lambda i: (i, 0)),
            ],
            out_specs=[],
            core_axis_name="s",
            dimension_semantics=(pltpu.PARALLEL,),
        )

        @pl.when(cid == 0)
        def _():
            pltpu.emit_pipeline(addmsg, **mspecs)(col_hbm, m0_hbm)

        @pl.when(cid == 1)
        def _():
            pltpu.emit_pipeline(addmsg, **mspecs)(col_hbm, m1_hbm)

        plsc.subcore_barrier()

        @pl.when(cid == 0)
        def _():
            _writeback(acc, s0_hbm, sid)

        @pl.when(cid == 1)
        def _():
            _writeback(acc, s1_hbm, sid)

    return k(msg0, msg1, col2d, zeros)


def _sc_counts(col_lo, col_hi, zeros, ones):
    @functools.partial(
        pl.kernel,
        out_type=(
            jax.ShapeDtypeStruct((NODE_PAD, D), jnp.float32),
            jax.ShapeDtypeStruct((NODE_PAD, D), jnp.float32),
        ),
        mesh=_mesh(),
        scratch_types=[
            pltpu.VMEM_SHARED((NODE_PAD, D), jnp.float32),
            pltpu.VMEM((W, D), jnp.float32),
        ],
    )
    def k(clo_hbm, chi_hbm, z_hbm, o_hbm, c0_hbm, c1_hbm, acc, obuf):
        cid = lax.axis_index("c")
        sid = lax.axis_index("s")

        _zero_acc(z_hbm, acc, sid)
        pltpu.sync_copy(o_hbm, obuf)
        plsc.subcore_barrier()

        def addones(i_v):
            pltpu.sync_copy(obuf, acc.at[i_v.at[0]], add=True)

        ospecs = dict(
            grid=(EDGE_GRID // 2,),
            in_specs=[pl.BlockSpec((1, W), lambda i: (0, i))],
            out_specs=[],
            core_axis_name="s",
            dimension_semantics=(pltpu.PARALLEL,),
        )

        @pl.when(cid == 0)
        def _():
            pltpu.emit_pipeline(addones, **ospecs)(clo_hbm)

        @pl.when(cid == 1)
        def _():
            pltpu.emit_pipeline(addones, **ospecs)(chi_hbm)

        plsc.subcore_barrier()

        @pl.when(cid == 0)
        def _():
            _writeback(acc, c0_hbm, sid)

        @pl.when(cid == 1)
        def _():
            _writeback(acc, c1_hbm, sid)

    return k(col_lo, col_hi, zeros, ones)


def _ln(x, g, b):
    mu = jnp.mean(x, axis=-1, keepdims=True)
    var = jnp.mean((x - mu) ** 2, axis=-1, keepdims=True)
    return (x - mu) / jnp.sqrt(var + 1e-5) * g + b


def _dot(a, b):
    return jnp.dot(a, b, preferred_element_type=jnp.float32)


def _tc_project(node_attr, w1a, w1b, v1a, relt_pad, w1c):
    def body(na_ref, wa_ref, wb_ref, va_ref, rt_ref, wc_ref,
             ac_ref, b_ref, rw_ref):
        na = na_ref[...]
        a = _dot(na, wa_ref[...])
        c = _dot(na, va_ref[...])
        au = lax.bitcast_convert_type(a, jnp.uint32)
        cu = lax.bitcast_convert_type(c, jnp.uint32)
        au = au + jnp.uint32(0x7FFF) + ((au >> 16) & jnp.uint32(1))
        cu = cu + jnp.uint32(0x7FFF) + ((cu >> 16) & jnp.uint32(1))
        packed = (au >> 16) | (cu & jnp.uint32(0xFFFF0000))
        ac_ref[...] = lax.bitcast_convert_type(packed, jnp.int32)
        b_ref[...] = _dot(na, wb_ref[...])
        rw_ref[...] = _dot(rt_ref[...], wc_ref[...]).astype(jnp.bfloat16)

    return pl.pallas_call(
        body,
        out_shape=(
            jax.ShapeDtypeStruct((NODE_PAD, D), jnp.int32),
            jax.ShapeDtypeStruct((NODE_PAD, D), jnp.float32),
            jax.ShapeDtypeStruct((R_PAD, D), jnp.bfloat16),
        ),
    )(node_attr, w1a, w1b, v1a, relt_pad, w1c)


def _tc_edge(ga, gb, rid, relw, ew):
    def body(ga_ref, gb_ref, rid_ref, rw_ref,
             eb1_r, eg1_r, ebe1_r, ew2_r, eb2_r, eg2_r, ebe2_r,
             ew3_r, eb3_r, v1b_r, nb1_r, ng1_r, nbe1_r, nv2_r, nb2_r,
             ng2_r, nbe2_r, nv3_r, nb3_r, eo_ref, m0_ref, m1_ref):
        bf = jnp.bfloat16
        oh = (rid_ref[...] == lax.broadcasted_iota(
            jnp.int32, (BE, R_PAD), 1)).astype(bf)
        gi = ga_ref[...]
        a = lax.bitcast_convert_type(gi << 16, jnp.float32)
        c_ = lax.bitcast_convert_type(gi & jnp.int32(-65536), jnp.float32)
        h = a + gb_ref[...] + _dot(oh, rw_ref[...]) + eb1_r[...]
        h = jax.nn.gelu(_ln(h, eg1_r[...], ebe1_r[...]))
        h = _dot(h.astype(bf), ew2_r[...]) + eb2_r[...]
        h = jax.nn.gelu(_ln(h, eg2_r[...], ebe2_r[...]))
        e = _dot(h.astype(bf), ew3_r[...]) + eb3_r[...]
        eo_ref[...] = e
        m = c_ + _dot(e.astype(bf), v1b_r[...]) + nb1_r[...]
        m = jax.nn.gelu(_ln(m, ng1_r[...], nbe1_r[...]))
        m = _dot(m.astype(bf), nv2_r[...]) + nb2_r[...]
        m = jax.nn.gelu(_ln(m, ng2_r[...], nbe2_r[...]))
        mb = m.astype(bf)
        m0_ref[...] = _dot(mb, nv3_r[:, :D]) + nb3_r[:, :D]
        m1_ref[...] = _dot(mb, nv3_r[:, D:]) + nb3_r[:, D:]

    full = lambda shape: pl.BlockSpec(shape, lambda i: tuple(0 for _ in shape))
    wspecs = [full(w.shape) for w in ew]
    return pl.pallas_call(
        body,
        grid=(N_EDGES // BE,),
        in_specs=[
            pl.BlockSpec((BE, D), lambda i: (i, 0)),
            pl.BlockSpec((BE, D), lambda i: (i, 0)),
            pl.BlockSpec((BE, 1), lambda i: (i, 0)),
            full((R_PAD, D)),
        ] + wspecs,
        out_specs=[
            pl.BlockSpec((BE, D), lambda i: (i, 0)),
            pl.BlockSpec((BE, D), lambda i: (i, 0)),
            pl.BlockSpec((BE, D), lambda i: (i, 0)),
        ],
        out_shape=(
            jax.ShapeDtypeStruct((N_EDGES, D), jnp.float32),
            jax.ShapeDtypeStruct((EP, D), jnp.float32),
            jax.ShapeDtypeStruct((EP, D), jnp.float32),
        ),
    )(ga, gb, rid, relw, *ew)


def _tc_node(node_attr, s0, s1, c0, c1, nw):
    def body(na_ref, s0_ref, s1_ref, c0_ref, c1_ref,
             u1a_r, u1b0_r, u1b1_r, ub1_r, ug1_r, ube1_r,
             u2_r, ub2_r, ug2_r, ube2_r, u3_r, ub3_r, o_ref):
        c = jnp.clip(c0_ref[...][:, 0:1] + c1_ref[...][:, 0:1], 1.0, None)
        m0 = s0_ref[...] / c
        m1 = s1_ref[...] / c
        h = (_dot(na_ref[...], u1a_r[...]) + _dot(m0, u1b0_r[...])
             + _dot(m1, u1b1_r[...]) + ub1_r[...])
        h = jax.nn.gelu(_ln(h, ug1_r[...], ube1_r[...]))
        h = _dot(h, u2_r[...]) + ub2_r[...]
        h = jax.nn.gelu(_ln(h, ug2_r[...], ube2_r[...]))
        o_ref[...] = _dot(h, u3_r[...]) + ub3_r[...]

    return pl.pallas_call(
        body,
        out_shape=jax.ShapeDtypeStruct((NODE_PAD, D), jnp.float32),
    )(node_attr, s0, s1, c0, c1, *nw)


def kernel(edge_index, c_ids, u, batch, num_tuples, rel_ids,
           concept_table, rel_table, edge_mlp, node_mlp1, node_mlp2):
    row = edge_index[0]
    col = edge_index[1]

    (eW1, eb1, eg1, ebe1), (eW2, eb2, eg2, ebe2), (eW3, eb3, _, _) = edge_mlp
    (nW1, nb1, ng1, nbe1), (nW2, nb2, ng2, nbe2), (nW3, nb3, _, _) = node_mlp1
    (uW1, ub1, ug1, ube1), (uW2, ub2, ug2, ube2), (uW3, ub3, _, _) = node_mlp2

    r2 = lambda v: v.reshape(1, -1)
    epad = lambda v, c: jnp.pad(v, (0, EP - N_EDGES),
                                constant_values=c).reshape(1, EP)

    c_pad = jnp.pad(c_ids, (0, C_PAD - N_NODES)).reshape(1, C_PAD)
    na_pad = _sc_embed_gather(concept_table, c_pad)
    node_attr = na_pad[:NODE_PAD]

    relt_pad = jnp.pad(rel_table, ((0, R_PAD - rel_table.shape[0]), (0, 0)))
    ac, bd, relw = _tc_project(node_attr, eW1[:D], eW1[D:2 * D], nW1[:D],
                               relt_pad, eW1[2 * D:])
    row2d = epad(row, DUMMY)
    col2d = epad(col, DUMMY)
    ga, gb = _sc_edge_gather(ac, bd, row2d, col2d)

    bf = lambda v: v.astype(jnp.bfloat16)
    ew = (r2(eb1), r2(eg1), r2(ebe1), bf(eW2), r2(eb2), r2(eg2),
          r2(ebe2), bf(eW3), r2(eb3), bf(nW1[D:]), r2(nb1), r2(ng1),
          r2(nbe1), bf(nW2), r2(nb2), r2(ng2), r2(nbe2), bf(nW3), r2(nb3))
    eout, m0, m1 = _tc_edge(ga, gb, rel_ids.reshape(N_EDGES, 1), relw, ew)

    zeros = jnp.zeros((W, D), jnp.float32)
    c0, c1 = _sc_counts(col2d[:, :EP // 2], col2d[:, EP // 2:],
                        zeros, jnp.ones((W, D), jnp.float32))
    s0, s1 = _sc_scatter_msg(m0, m1, col2d, zeros)

    nw = (uW1[:D], uW1[D:2 * D], uW1[2 * D:], r2(ub1), r2(ug1), r2(ube1),
          uW2, r2(ub2), r2(ug2), r2(ube2), uW3, r2(ub3))
    x_new = _tc_node(node_attr, s0, s1, c0, c1, nw)

    return (x_new[:N_NODES], eout, u)
